# initial kernel scaffold (unmeasured)
import jax
import jax.numpy as jnp
from jax import lax
from jax.experimental import pallas as pl
from jax.experimental.pallas import tpu as pltpu


def kernel(
    x,
):
    def body(*refs):
        pass

    out_shape = jax.ShapeDtypeStruct(..., jnp.float32)
    return pl.pallas_call(body, out_shape=out_shape)(...)


# baseline (device time: 17164 ns/iter reference)
import jax
import jax.numpy as jnp
from jax import lax
from jax.experimental import pallas as pl
from jax.experimental.pallas import tpu as pltpu


def kernel(x):
    m, n = x.shape

    def body(x_ref, out_ref, comm_ref, send_sem, recv_sem):
        my_x = lax.axis_index("x")
        my_y = lax.axis_index("y")
        ynbr = (my_x, 1 - my_y)

        barrier_sem = pltpu.get_barrier_semaphore()
        pl.semaphore_signal(
            barrier_sem, inc=1,
            device_id=ynbr, device_id_type=pl.DeviceIdType.MESH,
        )
        pl.semaphore_wait(barrier_sem, 1)

        rdma = pltpu.make_async_remote_copy(
            src_ref=x_ref,
            dst_ref=comm_ref,
            send_sem=send_sem,
            recv_sem=recv_sem,
            device_id=ynbr,
            device_id_type=pl.DeviceIdType.MESH,
        )
        rdma.start()
        rdma.wait()

        out_ref[...] = x_ref[...] + comm_ref[...]

    return pl.pallas_call(
        body,
        out_shape=jax.ShapeDtypeStruct((m, n), jnp.float32),
        in_specs=[pl.BlockSpec(memory_space=pltpu.VMEM)],
        out_specs=pl.BlockSpec(memory_space=pltpu.VMEM),
        scratch_shapes=[
            pltpu.VMEM((m, n), jnp.float32),
            pltpu.SemaphoreType.DMA,
            pltpu.SemaphoreType.DMA,
        ],
        compiler_params=pltpu.CompilerParams(collective_id=0),
    )(x)


# device time: 14910 ns/iter; 1.1512x vs baseline; 1.1512x over previous
import jax
import jax.numpy as jnp
from jax import lax
from jax.experimental import pallas as pl
from jax.experimental.pallas import tpu as pltpu

C = 8


def kernel(x):
    m, n = x.shape
    half = m // 2
    ch = half // C

    def body(x_ref, out_ref, comm_ref, s1, r1, s2, r2):
        my_x = lax.axis_index("x")
        my_y = lax.axis_index("y")
        ynbr = (my_x, 1 - my_y)
        xnbr = (1 - my_x, my_y)
        base = my_x * half

        barrier_sem = pltpu.get_barrier_semaphore()
        for nbr in (ynbr, xnbr):
            pl.semaphore_signal(
                barrier_sem, inc=1,
                device_id=nbr, device_id_type=pl.DeviceIdType.MESH,
            )
        pl.semaphore_wait(barrier_sem, 2)

        p1 = []
        for c in range(C):
            rdma = pltpu.make_async_remote_copy(
                src_ref=x_ref.at[pl.ds(base + c * ch, ch), :],
                dst_ref=comm_ref.at[pl.ds(c * ch, ch), :],
                send_sem=s1.at[c],
                recv_sem=r1.at[c],
                device_id=ynbr,
                device_id_type=pl.DeviceIdType.MESH,
            )
            rdma.start()
            p1.append(rdma)

        p2 = []
        for c in range(C):
            p1[c].wait_recv()
            sl = pl.ds(base + c * ch, ch)
            csl = pl.ds(c * ch, ch)
            out_ref[sl, :] = x_ref[sl, :] + comm_ref[csl, :]
            rdma = pltpu.make_async_remote_copy(
                src_ref=out_ref.at[sl, :],
                dst_ref=out_ref.at[sl, :],
                send_sem=s2.at[c],
                recv_sem=r2.at[c],
                device_id=xnbr,
                device_id_type=pl.DeviceIdType.MESH,
            )
            rdma.start()
            p2.append(rdma)

        for c in range(C):
            p1[c].wait_send()
            p2[c].wait()

    return pl.pallas_call(
        body,
        out_shape=jax.ShapeDtypeStruct((m, n), jnp.float32),
        in_specs=[pl.BlockSpec(memory_space=pltpu.VMEM)],
        out_specs=pl.BlockSpec(memory_space=pltpu.VMEM),
        scratch_shapes=[
            pltpu.VMEM((half, n), jnp.float32),
            pltpu.SemaphoreType.DMA((C,)),
            pltpu.SemaphoreType.DMA((C,)),
            pltpu.SemaphoreType.DMA((C,)),
            pltpu.SemaphoreType.DMA((C,)),
        ],
        compiler_params=pltpu.CompilerParams(collective_id=0),
    )(x)


# device time: 14812 ns/iter; 1.1588x vs baseline; 1.0066x over previous
import jax
import jax.numpy as jnp
from jax import lax
from jax.experimental import pallas as pl
from jax.experimental.pallas import tpu as pltpu

C = 16


def kernel(x):
    m, n = x.shape
    half = m // 2
    ch = half // C

    def body(x_ref, out_ref, comm_ref, s1, r1, s2, r2):
        my_x = lax.axis_index("x")
        my_y = lax.axis_index("y")
        ynbr = (my_x, 1 - my_y)
        xnbr = (1 - my_x, my_y)
        base = my_x * half

        barrier_sem = pltpu.get_barrier_semaphore()
        for nbr in (ynbr, xnbr):
            pl.semaphore_signal(
                barrier_sem, inc=1,
                device_id=nbr, device_id_type=pl.DeviceIdType.MESH,
            )
        pl.semaphore_wait(barrier_sem, 2)

        p1 = []
        for c in range(C):
            rdma = pltpu.make_async_remote_copy(
                src_ref=x_ref.at[pl.ds(base + c * ch, ch), :],
                dst_ref=comm_ref.at[pl.ds(c * ch, ch), :],
                send_sem=s1.at[c],
                recv_sem=r1.at[c],
                device_id=ynbr,
                device_id_type=pl.DeviceIdType.MESH,
            )
            rdma.start()
            p1.append(rdma)

        p2 = []
        for c in range(C):
            p1[c].wait_recv()
            sl = pl.ds(base + c * ch, ch)
            csl = pl.ds(c * ch, ch)
            out_ref[sl, :] = x_ref[sl, :] + comm_ref[csl, :]
            rdma = pltpu.make_async_remote_copy(
                src_ref=out_ref.at[sl, :],
                dst_ref=out_ref.at[sl, :],
                send_sem=s2.at[c],
                recv_sem=r2.at[c],
                device_id=xnbr,
                device_id_type=pl.DeviceIdType.MESH,
            )
            rdma.start()
            p2.append(rdma)

        for c in range(C):
            p1[c].wait_send()
            p2[c].wait()

    return pl.pallas_call(
        body,
        out_shape=jax.ShapeDtypeStruct((m, n), jnp.float32),
        in_specs=[pl.BlockSpec(memory_space=pltpu.VMEM)],
        out_specs=pl.BlockSpec(memory_space=pltpu.VMEM),
        scratch_shapes=[
            pltpu.VMEM((half, n), jnp.float32),
            pltpu.SemaphoreType.DMA((C,)),
            pltpu.SemaphoreType.DMA((C,)),
            pltpu.SemaphoreType.DMA((C,)),
            pltpu.SemaphoreType.DMA((C,)),
        ],
        compiler_params=pltpu.CompilerParams(collective_id=0),
    )(x)
